# baseline (speedup 1.0000x reference)
import functools

import jax
import jax.numpy as jnp
import numpy as np
from jax import lax
from jax.experimental import pallas as pl
from jax.experimental.pallas import tpu as pltpu
from jax.experimental.pallas import tpu_sc as plsc

N_NODES = 10000
N_EDGES = 320000
G = 64
D = 128
H1 = 128
H2 = 64
NCLS = 8

NW = 32
EPT = N_EDGES // NW
CHK = 40
NCHUNK = EPT // CHK
N_PAD = 10240
ROWS_PT = N_PAD // 16

BLK = 1000
NBLK = N_NODES // BLK

_HI = lax.Precision.HIGHEST
_C00 = (((0,), (0,)), ((), ()))

_PLO = np.empty(D // 2, np.int32)
_PHI = np.empty(D // 2, np.int32)
for _j in range(D // 32):
    for _t in range(16):
        _PLO[16 * _j + _t] = 32 * _j + _t
        _PHI[16 * _j + _t] = 32 * _j + 16 + _t
_KPERM = np.concatenate([_PLO, _PHI])
_PK_MAT = np.zeros((D, D), np.float32)
_PK_MAT[_KPERM, np.arange(D)] = 1.0


def _pack2(lo, hi):
    li = lax.convert_element_type(
        lax.bitcast_convert_type(lo.astype(jnp.bfloat16), jnp.uint16),
        jnp.int32)
    hh = lax.convert_element_type(
        lax.bitcast_convert_type(hi.astype(jnp.bfloat16), jnp.uint16),
        jnp.int32)
    return li | (hh << 16)


def _mm(a, b):
    return jnp.dot(a, b, preferred_element_type=jnp.float32, precision=_HI)


def _gelu(x):
    return x * 0.5 * (1.0 + lax.erf(x * (2.0 ** -0.5)))


def _projpack(h, w2, b2):
    hd = D // 2
    return _pack2(_mm(h, w2[:, :hd]) + b2[:, :hd],
                  _mm(h, w2[:, hd:]) + b2[:, hd:])


def _proj_outs(h, wk, bk, wq, bq, wv, bv, ws, bs, ko, qvo, so):
    ko[...] = _mm(h, wk) + bk
    qvo[...] = jnp.concatenate(
        [_projpack(h, wq[:, :D], bq[:, :D]),
         _projpack(h, wv[:, :D], bv[:, :D])], axis=1)
    so[...] = _mm(h, ws) + bs


def _lohi_w(W):
    return jnp.concatenate([W[:, _PLO], W[:, _PHI]], axis=1)


def _lohi_b(b):
    return jnp.concatenate([b[_PLO], b[_PHI]]).reshape(1, D)


def _proj4_body(h_ref, wk, bk, wq, bq, wv, bv, ws, bs, ko, qvo, so):
    _proj_outs(h_ref[...], wk[...], bk[...], wq[...], bq[...],
               wv[...], bv[...], ws[...], bs[...], ko, qvo, so)


def _proj4(h, Wk, bk, Wq, bq, Wv, bv, Ws, bs):
    blk = 2000
    row = pl.BlockSpec((blk, D), lambda i: (i, 0))
    irow = pl.BlockSpec((blk, D), lambda i: (i, 0))
    wsp = pl.BlockSpec((D, D), lambda i: (0, 0))
    bsp = pl.BlockSpec((1, D), lambda i: (0, 0))
    return pl.pallas_call(
        _proj4_body,
        grid=(N_NODES // blk,),
        in_specs=[row, wsp, bsp, wsp, bsp, wsp, bsp, wsp, bsp],
        out_specs=[row, irow, row],
        out_shape=[jax.ShapeDtypeStruct((N_NODES, D), jnp.float32),
                   jax.ShapeDtypeStruct((N_NODES, D), jnp.int32),
                   jax.ShapeDtypeStruct((N_NODES, D), jnp.float32)],
    )(h, Wk[:, _KPERM], bk[_KPERM].reshape(1, D), _lohi_w(Wq), _lohi_b(bq),
      _lohi_w(Wv), _lohi_b(bv), Ws, bs.reshape(1, D))


def _edge_body(k_hbm, qv_hbm, src_hbm, dst_hbm, out_hbm,
               is0, is1, is2, is3, id0, id1, id2, id3,
               sd0, kr0, qvr0, sb0, sd1, kr1, qvr1, sb1,
               zbuf, acc,
               semi0, semi1, semi2, semi3, semg0, semg1, semw0, semw1):
    c = lax.axis_index("c")
    s = lax.axis_index("s")
    wid = c * 16 + s
    base = wid * EPT
    last = NCHUNK - 1

    ring = ((is0, id0, semi0), (is1, id1, semi1),
            (is2, id2, semi2), (is3, id3, semi3))
    slots = ((sd0, kr0, qvr0, sb0, semg0, semw0),
             (sd1, kr1, qvr1, sb1, semg1, semw1))

    def idx_fetch(i, x):
        iss, idd, semi = ring[x]
        off = jnp.minimum(i, last) * CHK + base
        pltpu.async_copy(src_hbm.at[pl.ds(off, CHK)], iss, semi)
        pltpu.async_copy(dst_hbm.at[pl.ds(off, CHK)], idd, semi)

    def idx_wait(x):
        iss, idd, semi = ring[x]
        pltpu.make_async_copy(src_hbm.at[pl.ds(base, CHK)], iss, semi).wait()
        pltpu.make_async_copy(dst_hbm.at[pl.ds(base, CHK)], idd, semi).wait()

    def fire_gathers(b, x):
        iss, idd, _ = ring[x]
        sd, kr, qvr, sb, semg, semw = slots[b]
        idx_wait(x)
        pltpu.async_copy(k_hbm.at[idd], kr, semg)
        pltpu.async_copy(qv_hbm.at[iss], qvr, semg)

    def wait_gathers(b, x):
        iss, idd, _ = ring[x]
        sd, kr, qvr, sb, semg, semw = slots[b]
        pltpu.make_async_copy(k_hbm.at[idd], kr, semg).wait()
        pltpu.make_async_copy(qv_hbm.at[iss], qvr, semg).wait()

    def process(b, x, first):
        iss, idd, _ = ring[x]
        sd, kr, qvr, sb, semg, semw = slots[b]
        wait_gathers(b, x)
        if not first:
            pltpu.make_async_copy(sb, acc.at[sd], semw).wait()

        hi_mask = jnp.int32(-65536)

        @plsc.parallel_loop(0, CHK, 1, unroll=3)
        def _edge(e):
            for j in range(D // 32):
                ka = kr[e, pl.ds(16 * j, 16)]
                kb = kr[e, pl.ds(64 + 16 * j, 16)]
                q32 = qvr[e, pl.ds(16 * j, 16)]
                v32 = qvr[e, pl.ds(64 + 16 * j, 16)]
                qa = lax.bitcast_convert_type(q32 << 16, jnp.float32)
                qb = lax.bitcast_convert_type(q32 & hi_mask, jnp.float32)
                va = lax.bitcast_convert_type(v32 << 16, jnp.float32)
                vb = lax.bitcast_convert_type(v32 & hi_mask, jnp.float32)
                sb[e, pl.ds(j * 32, 16)] = va / (1.0 + jnp.exp(-(ka + qa)))
                sb[e, pl.ds(j * 32 + 16, 16)] = (
                    vb / (1.0 + jnp.exp(-(kb + qb))))
        for o in (0, 16, CHK - 16):
            sd[pl.ds(o, 16)] = idd[pl.ds(o, 16)]
        pltpu.async_copy(sb, acc.at[sd], semw, add=True)

    def drain_scatter(b):
        sd, kr, qvr, sb, semg, semw = slots[b]
        pltpu.make_async_copy(sb, acc.at[sd], semw).wait()

    for i in range(4):
        idx_fetch(i, i)

    def _zrow(r, carry):
        for j in range(D // 16):
            zbuf[r, pl.ds(j * 16, 16)] = jnp.zeros((16,), jnp.float32)
        return carry
    lax.fori_loop(0, 40, _zrow, 0)
    for t in range(ROWS_PT // 40):
        pltpu.sync_copy(zbuf, acc.at[pl.ds(s * ROWS_PT + t * 40, 40)])
    plsc.subcore_barrier()

    fire_gathers(0, 0)
    fire_gathers(1, 1)
    process(0, 0, True)
    idx_fetch(4, 0)
    fire_gathers(0, 2)
    process(1, 1, True)
    idx_fetch(5, 1)
    fire_gathers(1, 3)

    def _quad(iq, carry):
        c0 = 4 * iq + 2
        for m in range(4):
            x = (2 + m) % 4
            process(m % 2, x, False)
            idx_fetch(c0 + m + 4, x)
            fire_gathers(m % 2, (x + 2) % 4)
        return carry
    lax.fori_loop(0, (NCHUNK - 2) // 4, _quad, 0)

    wait_gathers(0, 2)
    wait_gathers(1, 3)
    idx_wait(0)
    idx_wait(1)
    drain_scatter(0)
    drain_scatter(1)

    plsc.subcore_barrier()
    r0 = s * ROWS_PT
    pltpu.sync_copy(acc.at[pl.ds(r0, ROWS_PT)],
                    out_hbm.at[c, pl.ds(r0, ROWS_PT)])


_edge_agg = functools.partial(
    pl.kernel,
    out_type=jax.ShapeDtypeStruct((2, N_PAD, D), jnp.float32),
    mesh=plsc.VectorSubcoreMesh(core_axis_name="c", subcore_axis_name="s"),
    scratch_types=(
        [pltpu.VMEM((CHK,), jnp.int32)] * 8
        + [pltpu.VMEM((CHK,), jnp.int32)]
        + [pltpu.VMEM((CHK, D), jnp.float32)]
        + [pltpu.VMEM((CHK, D), jnp.int32)]
        + [pltpu.VMEM((CHK, D), jnp.float32)]
        + [pltpu.VMEM((CHK,), jnp.int32)]
        + [pltpu.VMEM((CHK, D), jnp.float32)]
        + [pltpu.VMEM((CHK, D), jnp.int32)]
        + [pltpu.VMEM((CHK, D), jnp.float32)]
        + [
            pltpu.VMEM((40, D), jnp.float32),
            pltpu.VMEM_SHARED((N_PAD, D), jnp.float32),
        ]
        + [pltpu.SemaphoreType.DMA] * 8
    ),
)(_edge_body)


def _onehot(b2d, rows):
    P = (b2d[...] == lax.broadcasted_iota(jnp.int32, (rows, G), 1))
    return P.astype(jnp.float32)


def _stats_accum(i, P, hg, s1, s2, cnt):
    @pl.when(i == 0)
    def _():
        s1[...] = jnp.zeros((G, D), jnp.float32)
        s2[...] = jnp.zeros((G, D), jnp.float32)
        cnt[...] = jnp.zeros((G, D), jnp.float32)
    s1[...] += lax.dot_general(P, hg, _C00, precision=_HI)
    s2[...] += lax.dot_general(P, hg * hg, _C00, precision=_HI)
    cnt[...] += lax.dot_general(P, jnp.ones_like(hg), _C00, precision=_HI)


def _stats0_body(p0, p1, sk, b2d, hg_out, s1, s2, cnt):
    i = pl.program_id(0)
    hg = _gelu(p0[0] + p1[0] + sk[...])
    hg_out[...] = hg
    _stats_accum(i, _onehot(b2d, BLK), hg, s1, s2, cnt)


def _stats1_body(p0, p1, sk, b2d, s1, s2, cnt):
    i = pl.program_id(0)
    hg = _gelu(p0[0] + p1[0] + sk[...])
    _stats_accum(i, _onehot(b2d, BLK), hg, s1, s2, cnt)


def _stats_specs():
    prow = lambda core: pl.BlockSpec((1, BLK, D), lambda i, c=core: (c, i, 0))
    row = pl.BlockSpec((BLK, D), lambda i: (i, 0))
    bsp = pl.BlockSpec((BLK, 1), lambda i: (i, 0))
    gsp = pl.BlockSpec((G, D), lambda i: (0, 0))
    return prow, row, bsp, gsp


def _stats0(p, sk, b2d):
    prow, row, bsp, gsp = _stats_specs()
    return pl.pallas_call(
        _stats0_body,
        grid=(NBLK,),
        in_specs=[prow(0), prow(1), row, bsp],
        out_specs=[row, gsp, gsp, gsp],
        out_shape=[jax.ShapeDtypeStruct((N_NODES, D), jnp.float32)]
        + [jax.ShapeDtypeStruct((G, D), jnp.float32)] * 3,
    )(p, p, sk, b2d)


def _stats1(p, sk, b2d):
    prow, row, bsp, gsp = _stats_specs()
    return pl.pallas_call(
        _stats1_body,
        grid=(NBLK,),
        in_specs=[prow(0), prow(1), row, bsp],
        out_specs=[gsp, gsp, gsp],
        out_shape=[jax.ShapeDtypeStruct((G, D), jnp.float32)] * 3,
    )(p, p, sk, b2d)


def _gn_moments(s1, s2, cnt, gnm):
    c = jnp.maximum(cnt, 1.0)
    mg = s1 / c * gnm
    var = (s2 - 2.0 * mg * s1 + c * mg * mg) / c
    return mg, var, c


def _normproj_body(hg, s1, s2, cnt, b2d, gnw, gnb, gnm,
                   wk, bk, wq, bq, wv, bv, ws, bs, ko, qvo, so):
    mg, var, _ = _gn_moments(s1[...], s2[...], cnt[...], gnm[...])
    P = _onehot(b2d, BLK)
    o = hg[...] - _mm(P, mg)
    h = o * lax.rsqrt(_mm(P, var) + 1e-5) * gnw[...] + gnb[...]
    _proj_outs(h, wk[...], bk[...], wq[...], bq[...], wv[...],
               bv[...], ws[...], bs[...], ko, qvo, so)


def _normproj(hg, s1, s2, cnt, b2d, gnw, gnb, gnm,
              Wk, bk, Wq, bq, Wv, bv, Ws, bs):
    row = pl.BlockSpec((BLK, D), lambda i: (i, 0))
    bsp = pl.BlockSpec((BLK, 1), lambda i: (i, 0))
    gsp = pl.BlockSpec((G, D), lambda i: (0, 0))
    vsp = pl.BlockSpec((1, D), lambda i: (0, 0))
    wsp = pl.BlockSpec((D, D), lambda i: (0, 0))
    return pl.pallas_call(
        _normproj_body,
        grid=(NBLK,),
        in_specs=[row, gsp, gsp, gsp, bsp, vsp, vsp, vsp,
                  wsp, vsp, wsp, vsp, wsp, vsp, wsp, vsp],
        out_specs=[row, row, row],
        out_shape=[jax.ShapeDtypeStruct((N_NODES, D), jnp.float32),
                   jax.ShapeDtypeStruct((N_NODES, D), jnp.int32),
                   jax.ShapeDtypeStruct((N_NODES, D), jnp.float32)],
    )(hg, s1, s2, cnt, b2d, gnw.reshape(1, D), gnb.reshape(1, D),
      gnm.reshape(1, D), Wk, bk.reshape(1, D),
      Wq, bq.reshape(1, D), Wv, bv.reshape(1, D), Ws, bs.reshape(1, D))


def _ln(t, w, b):
    m = jnp.mean(t, axis=-1, keepdims=True)
    v = jnp.mean((t - m) ** 2, axis=-1, keepdims=True)
    return (t - m) * lax.rsqrt(v + 1e-5) * w[...] + b[...]


def _head_body(s1, s2, cnt, gnw, gnb, gnm,
               wh0, bh0, lnw0, lnb0, wh1, bh1, lnw1, lnb1, wl, bl, out):
    mg, var, c = _gn_moments(s1[...], s2[...], cnt[...], gnm[...])
    pooled = (s1[...] * (1.0 - gnm[...]) * lax.rsqrt(var + 1e-5) * gnw[...] / c
              + gnb[...])
    t = jax.nn.relu(_mm(pooled, wh0[...]) + bh0[...])
    t = _ln(t, lnw0, lnb0)
    t = jax.nn.relu(_mm(t, wh1[...]) + bh1[...])
    t = _ln(t, lnw1, lnb1)
    out[...] = _mm(t, wl[...]) + bl[...]


def _head(s1, s2, cnt, gnw, gnb, gnm,
          Wh0, bh0, lnw0, lnb0, Wh1, bh1, lnw1, lnb1, Wl, bl):
    gsp = pl.BlockSpec((G, D), lambda: (0, 0))

    def vec(n):
        return pl.BlockSpec((1, n), lambda: (0, 0))

    def mat(m, n):
        return pl.BlockSpec((m, n), lambda: (0, 0))

    return pl.pallas_call(
        _head_body,
        in_specs=[gsp, gsp, gsp, vec(D), vec(D), vec(D),
                  mat(D, H1), vec(H1), vec(H1), vec(H1),
                  mat(H1, H2), vec(H2), vec(H2), vec(H2),
                  mat(H2, NCLS), vec(NCLS)],
        out_specs=pl.BlockSpec((G, NCLS), lambda: (0, 0)),
        out_shape=jax.ShapeDtypeStruct((G, NCLS), jnp.float32),
    )(s1, s2, cnt, gnw.reshape(1, D), gnb.reshape(1, D), gnm.reshape(1, D),
      Wh0, bh0.reshape(1, H1), lnw0.reshape(1, H1), lnb0.reshape(1, H1),
      Wh1, bh1.reshape(1, H2), lnw1.reshape(1, H2), lnb1.reshape(1, H2),
      Wl, bl.reshape(1, NCLS))


def kernel(x, edge_index, batch,
           Wk0, bk0, Wq0, bq0, Wv0, bv0, Ws0, bs0, gnw0, gnb0, gnm0,
           Wk1, bk1, Wq1, bq1, Wv1, bv1, Ws1, bs1, gnw1, gnb1, gnm1,
           Wh0, bh0, lnw0, lnb0, Wh1, bh1, lnw1, lnb1, Wl, bl):
    b2d = batch.astype(jnp.int32).reshape(N_NODES, 1)
    ei = edge_index.astype(jnp.int32)
    src, dst = ei[0], ei[1]

    k0, qv0, s0 = _proj4(x, Wk0, bk0, Wq0, bq0, Wv0, bv0, Ws0, bs0)
    p = _edge_agg(k0, qv0, src, dst)
    hg0, s1, s2, cnt = _stats0(p, s0, b2d)
    k1, qv1, s1_ = _normproj(hg0, s1, s2, cnt, b2d, gnw0, gnb0, gnm0,
                             Wk1, bk1, Wq1, bq1, Wv1, bv1, Ws1, bs1)
    p = _edge_agg(k1, qv1, src, dst)
    t1, t2, tc = _stats1(p, s1_, b2d)
    return _head(t1, t2, tc, gnw1, gnb1, gnm1,
                 Wh0, bh0, lnw0, lnb0, Wh1, bh1, lnw1, lnb1, Wl, bl)


# speedup vs baseline: 1.0341x; 1.0341x over previous
import functools

import jax
import jax.numpy as jnp
import numpy as np
from jax import lax
from jax.experimental import pallas as pl
from jax.experimental.pallas import tpu as pltpu
from jax.experimental.pallas import tpu_sc as plsc

N_NODES = 10000
N_EDGES = 320000
G = 64
D = 128
H1 = 128
H2 = 64
NCLS = 8

NW = 32
EPT = N_EDGES // NW
CHK = 40
NCHUNK = EPT // CHK
N_PAD = 10240
ROWS_PT = N_PAD // 16

BLK = 1000
NBLK = N_NODES // BLK

_HI = lax.Precision.HIGHEST
_C00 = (((0,), (0,)), ((), ()))

_PLO = np.empty(D // 2, np.int32)
_PHI = np.empty(D // 2, np.int32)
for _j in range(D // 32):
    for _t in range(16):
        _PLO[16 * _j + _t] = 32 * _j + _t
        _PHI[16 * _j + _t] = 32 * _j + 16 + _t
_KPERM = np.concatenate([_PLO, _PHI])
_PK_MAT = np.zeros((D, D), np.float32)
_PK_MAT[_KPERM, np.arange(D)] = 1.0


def _pack2(lo, hi):
    li = lax.convert_element_type(
        lax.bitcast_convert_type(lo.astype(jnp.bfloat16), jnp.uint16),
        jnp.int32)
    hh = lax.convert_element_type(
        lax.bitcast_convert_type(hi.astype(jnp.bfloat16), jnp.uint16),
        jnp.int32)
    return li | (hh << 16)


def _mm(a, b):
    return jnp.dot(a, b, preferred_element_type=jnp.float32, precision=_HI)


def _gelu(x):
    return x * 0.5 * (1.0 + lax.erf(x * (2.0 ** -0.5)))


def _projpack(h, w2, b2):
    hd = D // 2
    return _pack2(_mm(h, w2[:, :hd]) + b2[:, :hd],
                  _mm(h, w2[:, hd:]) + b2[:, hd:])


def _proj_outs(h, wk, bk, wq, bq, wv, bv, ws, bs, ko, qvo, so):
    ko[...] = _mm(h, wk) + bk
    qvo[...] = jnp.concatenate(
        [_projpack(h, wq[:, :D], bq[:, :D]),
         _projpack(h, wv[:, :D], bv[:, :D])], axis=1)
    so[...] = _mm(h, ws) + bs


def _lohi_w(W):
    return jnp.concatenate([W[:, _PLO], W[:, _PHI]], axis=1)


def _lohi_b(b):
    return jnp.concatenate([b[_PLO], b[_PHI]]).reshape(1, D)


def _proj4_body(h_ref, wk, bk, wq, bq, wv, bv, ws, bs, ko, qvo, so):
    _proj_outs(h_ref[...], wk[...], bk[...], wq[...], bq[...],
               wv[...], bv[...], ws[...], bs[...], ko, qvo, so)


def _proj4(h, Wk, bk, Wq, bq, Wv, bv, Ws, bs):
    blk = 2000
    row = pl.BlockSpec((blk, D), lambda i: (i, 0))
    irow = pl.BlockSpec((blk, D), lambda i: (i, 0))
    wsp = pl.BlockSpec((D, D), lambda i: (0, 0))
    bsp = pl.BlockSpec((1, D), lambda i: (0, 0))
    return pl.pallas_call(
        _proj4_body,
        grid=(N_NODES // blk,),
        in_specs=[row, wsp, bsp, wsp, bsp, wsp, bsp, wsp, bsp],
        out_specs=[row, irow, row],
        out_shape=[jax.ShapeDtypeStruct((N_NODES, D), jnp.float32),
                   jax.ShapeDtypeStruct((N_NODES, D), jnp.int32),
                   jax.ShapeDtypeStruct((N_NODES, D), jnp.float32)],
    )(h, Wk[:, _KPERM], bk[_KPERM].reshape(1, D), _lohi_w(Wq), _lohi_b(bq),
      _lohi_w(Wv), _lohi_b(bv), Ws, bs.reshape(1, D))


def _edge_body(k_hbm, qv_hbm, src_hbm, dst_hbm, out_hbm,
               is0, is1, is2, is3, id0, id1, id2, id3,
               sd0, kr0, qvr0, sb0, sd1, kr1, qvr1, sb1,
               zbuf, acc,
               semi0, semi1, semi2, semi3, semg0, semg1, semw0, semw1):
    c = lax.axis_index("c")
    s = lax.axis_index("s")
    wid = c * 16 + s
    base = wid * EPT
    last = NCHUNK - 1

    ring = ((is0, id0, semi0), (is1, id1, semi1),
            (is2, id2, semi2), (is3, id3, semi3))
    slots = ((sd0, kr0, qvr0, sb0, semg0, semw0),
             (sd1, kr1, qvr1, sb1, semg1, semw1))

    def idx_fetch(i, x):
        iss, idd, semi = ring[x]
        off = jnp.minimum(i, last) * CHK + base
        pltpu.async_copy(src_hbm.at[pl.ds(off, CHK)], iss, semi)
        pltpu.async_copy(dst_hbm.at[pl.ds(off, CHK)], idd, semi)

    def idx_wait(x):
        iss, idd, semi = ring[x]
        pltpu.make_async_copy(src_hbm.at[pl.ds(base, CHK)], iss, semi).wait()
        pltpu.make_async_copy(dst_hbm.at[pl.ds(base, CHK)], idd, semi).wait()

    def fire_gathers(b, x):
        iss, idd, _ = ring[x]
        sd, kr, qvr, sb, semg, semw = slots[b]
        idx_wait(x)
        pltpu.async_copy(k_hbm.at[idd], kr, semg)
        pltpu.async_copy(qv_hbm.at[iss], qvr, semg)

    def wait_gathers(b, x):
        iss, idd, _ = ring[x]
        sd, kr, qvr, sb, semg, semw = slots[b]
        pltpu.make_async_copy(k_hbm.at[idd], kr, semg).wait()
        pltpu.make_async_copy(qv_hbm.at[iss], qvr, semg).wait()

    def process(b, x, first):
        iss, idd, _ = ring[x]
        sd, kr, qvr, sb, semg, semw = slots[b]
        wait_gathers(b, x)
        if not first:
            pltpu.make_async_copy(sb, acc.at[sd], semw).wait()

        hi_mask = jnp.int32(-65536)

        @plsc.parallel_loop(0, CHK, 1, unroll=2)
        def _edge(e):
            for j in range(D // 32):
                ka = kr[e, pl.ds(16 * j, 16)]
                kb = kr[e, pl.ds(64 + 16 * j, 16)]
                q32 = qvr[e, pl.ds(16 * j, 16)]
                v32 = qvr[e, pl.ds(64 + 16 * j, 16)]
                qa = lax.bitcast_convert_type(q32 << 16, jnp.float32)
                qb = lax.bitcast_convert_type(q32 & hi_mask, jnp.float32)
                va = lax.bitcast_convert_type(v32 << 16, jnp.float32)
                vb = lax.bitcast_convert_type(v32 & hi_mask, jnp.float32)
                sb[e, pl.ds(j * 32, 16)] = va / (1.0 + jnp.exp(-(ka + qa)))
                sb[e, pl.ds(j * 32 + 16, 16)] = (
                    vb / (1.0 + jnp.exp(-(kb + qb))))
        for o in (0, 16, CHK - 16):
            sd[pl.ds(o, 16)] = idd[pl.ds(o, 16)]
        pltpu.async_copy(sb, acc.at[sd], semw, add=True)

    def drain_scatter(b):
        sd, kr, qvr, sb, semg, semw = slots[b]
        pltpu.make_async_copy(sb, acc.at[sd], semw).wait()

    for i in range(4):
        idx_fetch(i, i)

    def _zrow(r, carry):
        for j in range(D // 16):
            zbuf[r, pl.ds(j * 16, 16)] = jnp.zeros((16,), jnp.float32)
        return carry
    lax.fori_loop(0, 40, _zrow, 0)
    for t in range(ROWS_PT // 40):
        pltpu.sync_copy(zbuf, acc.at[pl.ds(s * ROWS_PT + t * 40, 40)])
    plsc.subcore_barrier()

    fire_gathers(0, 0)
    fire_gathers(1, 1)
    process(0, 0, True)
    idx_fetch(4, 0)
    fire_gathers(0, 2)
    process(1, 1, True)
    idx_fetch(5, 1)
    fire_gathers(1, 3)

    def _quad(iq, carry):
        c0 = 4 * iq + 2
        for m in range(4):
            x = (2 + m) % 4
            process(m % 2, x, False)
            idx_fetch(c0 + m + 4, x)
            fire_gathers(m % 2, (x + 2) % 4)
        return carry
    lax.fori_loop(0, (NCHUNK - 2) // 4, _quad, 0)

    wait_gathers(0, 2)
    wait_gathers(1, 3)
    idx_wait(0)
    idx_wait(1)
    drain_scatter(0)
    drain_scatter(1)

    plsc.subcore_barrier()
    r0 = s * ROWS_PT
    pltpu.sync_copy(acc.at[pl.ds(r0, ROWS_PT)],
                    out_hbm.at[c, pl.ds(r0, ROWS_PT)])


_edge_agg = functools.partial(
    pl.kernel,
    out_type=jax.ShapeDtypeStruct((2, N_PAD, D), jnp.float32),
    mesh=plsc.VectorSubcoreMesh(core_axis_name="c", subcore_axis_name="s"),
    scratch_types=(
        [pltpu.VMEM((CHK,), jnp.int32)] * 8
        + [pltpu.VMEM((CHK,), jnp.int32)]
        + [pltpu.VMEM((CHK, D), jnp.float32)]
        + [pltpu.VMEM((CHK, D), jnp.int32)]
        + [pltpu.VMEM((CHK, D), jnp.float32)]
        + [pltpu.VMEM((CHK,), jnp.int32)]
        + [pltpu.VMEM((CHK, D), jnp.float32)]
        + [pltpu.VMEM((CHK, D), jnp.int32)]
        + [pltpu.VMEM((CHK, D), jnp.float32)]
        + [
            pltpu.VMEM((40, D), jnp.float32),
            pltpu.VMEM_SHARED((N_PAD, D), jnp.float32),
        ]
        + [pltpu.SemaphoreType.DMA] * 8
    ),
)(_edge_body)


def _onehot(b2d, rows):
    P = (b2d[...] == lax.broadcasted_iota(jnp.int32, (rows, G), 1))
    return P.astype(jnp.float32)


def _stats_accum(i, P, hg, s1, s2, cnt):
    @pl.when(i == 0)
    def _():
        s1[...] = jnp.zeros((G, D), jnp.float32)
        s2[...] = jnp.zeros((G, D), jnp.float32)
        cnt[...] = jnp.zeros((G, D), jnp.float32)
    s1[...] += lax.dot_general(P, hg, _C00, precision=_HI)
    s2[...] += lax.dot_general(P, hg * hg, _C00, precision=_HI)
    cnt[...] += lax.dot_general(P, jnp.ones_like(hg), _C00, precision=_HI)


def _stats0_body(p0, p1, sk, b2d, hg_out, s1, s2, cnt):
    i = pl.program_id(0)
    hg = _gelu(p0[0] + p1[0] + sk[...])
    hg_out[...] = hg
    _stats_accum(i, _onehot(b2d, BLK), hg, s1, s2, cnt)


def _stats_specs():
    prow = lambda core: pl.BlockSpec((1, BLK, D), lambda i, c=core: (c, i, 0))
    row = pl.BlockSpec((BLK, D), lambda i: (i, 0))
    bsp = pl.BlockSpec((BLK, 1), lambda i: (i, 0))
    gsp = pl.BlockSpec((G, D), lambda i: (0, 0))
    return prow, row, bsp, gsp


def _stats0(p, sk, b2d):
    prow, row, bsp, gsp = _stats_specs()
    return pl.pallas_call(
        _stats0_body,
        grid=(NBLK,),
        in_specs=[prow(0), prow(1), row, bsp],
        out_specs=[row, gsp, gsp, gsp],
        out_shape=[jax.ShapeDtypeStruct((N_NODES, D), jnp.float32)]
        + [jax.ShapeDtypeStruct((G, D), jnp.float32)] * 3,
    )(p, p, sk, b2d)


def _gn_moments(s1, s2, cnt, gnm):
    c = jnp.maximum(cnt, 1.0)
    mg = s1 / c * gnm
    var = (s2 - 2.0 * mg * s1 + c * mg * mg) / c
    return mg, var, c


def _normproj_body(hg, s1, s2, cnt, b2d, gnw, gnb, gnm,
                   wk, bk, wq, bq, wv, bv, ws, bs, ko, qvo, so):
    mg, var, _ = _gn_moments(s1[...], s2[...], cnt[...], gnm[...])
    P = _onehot(b2d, BLK)
    o = hg[...] - _mm(P, mg)
    h = o * lax.rsqrt(_mm(P, var) + 1e-5) * gnw[...] + gnb[...]
    _proj_outs(h, wk[...], bk[...], wq[...], bq[...], wv[...],
               bv[...], ws[...], bs[...], ko, qvo, so)


def _normproj(hg, s1, s2, cnt, b2d, gnw, gnb, gnm,
              Wk, bk, Wq, bq, Wv, bv, Ws, bs):
    row = pl.BlockSpec((BLK, D), lambda i: (i, 0))
    bsp = pl.BlockSpec((BLK, 1), lambda i: (i, 0))
    gsp = pl.BlockSpec((G, D), lambda i: (0, 0))
    vsp = pl.BlockSpec((1, D), lambda i: (0, 0))
    wsp = pl.BlockSpec((D, D), lambda i: (0, 0))
    return pl.pallas_call(
        _normproj_body,
        grid=(NBLK,),
        in_specs=[row, gsp, gsp, gsp, bsp, vsp, vsp, vsp,
                  wsp, vsp, wsp, vsp, wsp, vsp, wsp, vsp],
        out_specs=[row, row, row],
        out_shape=[jax.ShapeDtypeStruct((N_NODES, D), jnp.float32),
                   jax.ShapeDtypeStruct((N_NODES, D), jnp.int32),
                   jax.ShapeDtypeStruct((N_NODES, D), jnp.float32)],
    )(hg, s1, s2, cnt, b2d, gnw.reshape(1, D), gnb.reshape(1, D),
      gnm.reshape(1, D), Wk, bk.reshape(1, D),
      Wq, bq.reshape(1, D), Wv, bv.reshape(1, D), Ws, bs.reshape(1, D))


def _ln(t, w, b):
    m = jnp.mean(t, axis=-1, keepdims=True)
    v = jnp.mean((t - m) ** 2, axis=-1, keepdims=True)
    return (t - m) * lax.rsqrt(v + 1e-5) * w[...] + b[...]


def _stats_head_body(p0, p1, sk, b2d, gnw, gnb, gnm,
                     wh0, bh0, lnw0, lnb0, wh1, bh1, lnw1, lnb1, wl, bl,
                     out, s1, s2, cnt):
    i = pl.program_id(0)
    hg = _gelu(p0[0] + p1[0] + sk[...])
    _stats_accum(i, _onehot(b2d, BLK), hg, s1, s2, cnt)

    @pl.when(i == NBLK - 1)
    def _():
        mg, var, c = _gn_moments(s1[...], s2[...], cnt[...], gnm[...])
        pooled = (s1[...] * (1.0 - gnm[...]) * lax.rsqrt(var + 1e-5)
                  * gnw[...] / c + gnb[...])
        t = jax.nn.relu(_mm(pooled, wh0[...]) + bh0[...])
        t = _ln(t, lnw0, lnb0)
        t = jax.nn.relu(_mm(t, wh1[...]) + bh1[...])
        t = _ln(t, lnw1, lnb1)
        out[...] = _mm(t, wl[...]) + bl[...]


def _stats_head(p, sk, b2d, gnw, gnb, gnm,
                Wh0, bh0, lnw0, lnb0, Wh1, bh1, lnw1, lnb1, Wl, bl):
    prow, row, bsp, gsp = _stats_specs()

    def vec(n):
        return pl.BlockSpec((1, n), lambda i: (0, 0))

    def mat(m, n):
        return pl.BlockSpec((m, n), lambda i: (0, 0))

    return pl.pallas_call(
        _stats_head_body,
        grid=(NBLK,),
        in_specs=[prow(0), prow(1), row, bsp, vec(D), vec(D), vec(D),
                  mat(D, H1), vec(H1), vec(H1), vec(H1),
                  mat(H1, H2), vec(H2), vec(H2), vec(H2),
                  mat(H2, NCLS), vec(NCLS)],
        out_specs=pl.BlockSpec((G, NCLS), lambda i: (0, 0)),
        out_shape=jax.ShapeDtypeStruct((G, NCLS), jnp.float32),
        scratch_shapes=[pltpu.VMEM((G, D), jnp.float32)] * 3,
    )(p, p, sk, b2d, gnw.reshape(1, D), gnb.reshape(1, D), gnm.reshape(1, D),
      Wh0, bh0.reshape(1, H1), lnw0.reshape(1, H1), lnb0.reshape(1, H1),
      Wh1, bh1.reshape(1, H2), lnw1.reshape(1, H2), lnb1.reshape(1, H2),
      Wl, bl.reshape(1, NCLS))


def kernel(x, edge_index, batch,
           Wk0, bk0, Wq0, bq0, Wv0, bv0, Ws0, bs0, gnw0, gnb0, gnm0,
           Wk1, bk1, Wq1, bq1, Wv1, bv1, Ws1, bs1, gnw1, gnb1, gnm1,
           Wh0, bh0, lnw0, lnb0, Wh1, bh1, lnw1, lnb1, Wl, bl):
    b2d = batch.astype(jnp.int32).reshape(N_NODES, 1)
    ei = edge_index.astype(jnp.int32)
    src, dst = ei[0], ei[1]

    k0, qv0, s0 = _proj4(x, Wk0, bk0, Wq0, bq0, Wv0, bv0, Ws0, bs0)
    p = _edge_agg(k0, qv0, src, dst)
    hg0, s1, s2, cnt = _stats0(p, s0, b2d)
    k1, qv1, s1_ = _normproj(hg0, s1, s2, cnt, b2d, gnw0, gnb0, gnm0,
                             Wk1, bk1, Wq1, bq1, Wv1, bv1, Ws1, bs1)
    p = _edge_agg(k1, qv1, src, dst)
    return _stats_head(p, s1_, b2d, gnw1, gnb1, gnm1,
                       Wh0, bh0, lnw0, lnb0, Wh1, bh1, lnw1, lnb1, Wl, bl)
